# baseline (speedup 1.0000x reference)
import functools

import jax
import jax.numpy as jnp
from jax import lax
from jax.experimental import pallas as pl
from jax.experimental.pallas import tpu as pltpu
from jax.experimental.pallas import tpu_sc as plsc

B = 16384
F = 26
D = 16
FIELD = 38462
N = B * F

NC = 2
NS = 16
NW = NC * NS
EPW = N // NW
E = 1664
CHUNKS = EPW // E
LANES = 16


def _body(x_hbm, table_hbm, out_hbm, pat, xbuf, idxbuf, rows, sem):
    wid = lax.axis_index("s") * NC + lax.axis_index("c")
    base = wid * EPW

    def mk_pat(j, carry):
        pos = lax.iota(jnp.int32, LANES) + j * LANES
        pat[pl.ds(j * LANES, LANES)] = lax.rem(pos, F) * FIELD
        return carry

    lax.fori_loop(0, E // LANES, mk_pat, 0)

    def do_chunk(c, carry):
        off = base + c * E
        pltpu.sync_copy(x_hbm.at[pl.ds(off, E)], xbuf)

        def add_j(j, inner):
            s = pl.ds(j * LANES, LANES)
            idxbuf[s] = xbuf[s] + pat[s]
            return inner

        lax.fori_loop(0, E // LANES, add_j, 0)
        pltpu.async_copy(table_hbm.at[idxbuf], rows, sem).wait()
        pltpu.sync_copy(rows, out_hbm.at[pl.ds(off, E)])
        return carry

    lax.fori_loop(0, CHUNKS, do_chunk, 0)


@jax.jit
def _lookup(x_flat, table):
    mesh = plsc.VectorSubcoreMesh(
        core_axis_name="c", subcore_axis_name="s", num_cores=NC, num_subcores=NS
    )
    return pl.kernel(
        _body,
        out_type=jax.ShapeDtypeStruct((N, D), jnp.float32),
        mesh=mesh,
        compiler_params=pltpu.CompilerParams(use_tc_tiling_on_sc=False),
        scratch_types=[
            pltpu.VMEM((E,), jnp.int32),
            pltpu.VMEM((E,), jnp.int32),
            pltpu.VMEM((E,), jnp.int32),
            pltpu.VMEM((E, D), jnp.float32),
            pltpu.SemaphoreType.DMA,
        ],
    )(x_flat, table)


def kernel(x, table):
    out = _lookup(x.reshape(N), table)
    return out.reshape(B, F, D)


# speedup vs baseline: 1.3121x; 1.3121x over previous
import jax
import jax.numpy as jnp
from jax import lax
from jax.experimental import pallas as pl
from jax.experimental.pallas import tpu as pltpu
from jax.experimental.pallas import tpu_sc as plsc

B = 16384
F = 26
D = 16
FIELD = 38462

NC = 2
NS = 16
NW = NC * NS
BT = B // 128
UNITS = F * BT
UPW = UNITS // NW
LANES = 16


def _body(xt_hbm, table_hbm, out_hbm, idxbuf, rows, rowsT, sem):
    wid = lax.axis_index("s") * NC + lax.axis_index("c")
    base = wid * UPW
    dvec = lax.iota(jnp.int32, LANES) * 128

    def do_unit(i, carry):
        u = base + i
        f = u // BT
        bt = u % BT
        pltpu.sync_copy(xt_hbm.at[f, pl.ds(bt * 128, 128)], idxbuf)
        off = f * FIELD

        def add_j(j, inner):
            s = pl.ds(j * LANES, LANES)
            idxbuf[s] = idxbuf[s] + off
            return inner

        lax.fori_loop(0, 128 // LANES, add_j, 0)
        pltpu.async_copy(table_hbm.at[idxbuf], rows, sem).wait()

        def tr(bl, inner):
            vec = rows[bl, :]
            blv = jnp.full((LANES,), bl, jnp.int32)
            plsc.store_scatter(rowsT, [dvec + blv], vec)
            return inner

        lax.fori_loop(0, 128, tr, 0)
        pltpu.sync_copy(rowsT.at[pl.ds(0, 1024)], out_hbm.at[f, 0, bt])
        pltpu.sync_copy(rowsT.at[pl.ds(1024, 1024)], out_hbm.at[f, 1, bt])
        return carry

    lax.fori_loop(0, UPW, do_unit, 0)


@jax.jit
def _lookup(xt, table):
    mesh = plsc.VectorSubcoreMesh(
        core_axis_name="c", subcore_axis_name="s", num_cores=NC, num_subcores=NS
    )
    return pl.kernel(
        _body,
        out_type=jax.ShapeDtypeStruct((F, 2, BT, 1024), jnp.float32),
        mesh=mesh,
        compiler_params=pltpu.CompilerParams(
            use_tc_tiling_on_sc=False, needs_layout_passes=False
        ),
        scratch_types=[
            pltpu.VMEM((128,), jnp.int32),
            pltpu.VMEM((128, D), jnp.float32),
            pltpu.VMEM((D * 128,), jnp.float32),
            pltpu.SemaphoreType.DMA,
        ],
    )(xt, table)


def kernel(x, table):
    out5 = _lookup(x.T, table).reshape(F, 2, BT, 8, 128)
    return out5.transpose(2, 4, 0, 1, 3).reshape(B, F, D)


# speedup vs baseline: 1.5583x; 1.1877x over previous
import jax
import jax.numpy as jnp
from jax import lax
from jax.experimental import pallas as pl
from jax.experimental.pallas import tpu as pltpu
from jax.experimental.pallas import tpu_sc as plsc

B = 16384
F = 26
D = 16
FIELD = 38462

NC = 2
NS = 16
NW = NC * NS
BT = B // 128
UNITS = F * BT
UPW = UNITS // NW
G = 2
GPW = UPW // G
LANES = 16


def _unit_f_bt(base, g, j):
    u = base + g * G + j
    return u // BT, u % BT


def _prep_idx(idxall, idxg, base, g):
    for j in range(G):
        f, _ = _unit_f_bt(base, g, j)
        off = f * FIELD
        for j2 in range(128 // LANES):
            s = pl.ds(g * (G * 128) + j * 128 + j2 * LANES, LANES)
            d = pl.ds(j * 128 + j2 * LANES, LANES)
            idxg[d] = idxall[s] + off


def _transpose_unit(rows, j, idxT, rT):
    for bl in range(128):
        vec = rows[j * 128 + bl, :]
        ivec = idxT[pl.ds(bl * LANES, LANES)]
        plsc.store_scatter(rT, [ivec], vec)


def _start_writes(out_hbm, rT, base, g, j, sem):
    f, bt = _unit_f_bt(base, g, j)
    pltpu.make_async_copy(rT.at[pl.ds(0, 1024)], out_hbm.at[f, 0, bt], sem).start()
    pltpu.make_async_copy(rT.at[pl.ds(1024, 1024)], out_hbm.at[f, 1, bt], sem).start()


def _wait_writes(out_hbm, rT, base, g, j, sem):
    f, bt = _unit_f_bt(base, g, j)
    pltpu.make_async_copy(rT.at[pl.ds(0, 1024)], out_hbm.at[f, 0, bt], sem).wait()
    pltpu.make_async_copy(rT.at[pl.ds(1024, 1024)], out_hbm.at[f, 1, bt], sem).wait()


def _body(
    xt_hbm,
    table_hbm,
    out_hbm,
    idxall,
    idxT,
    idxg0,
    idxg1,
    rows0,
    rows1,
    rT00,
    rT01,
    rT10,
    rT11,
    sg0,
    sg1,
    sw00,
    sw01,
    sw10,
    sw11,
):
    wid = lax.axis_index("s") * NC + lax.axis_index("c")
    base = wid * UPW
    pltpu.sync_copy(xt_hbm.at[pl.ds(base * 128, UPW * 128)], idxall)
    dv = lax.iota(jnp.int32, LANES) * 128

    def mk_idx(j, c):
        idxT[pl.ds(j * LANES, LANES)] = dv + j
        return c

    lax.fori_loop(0, 128, mk_idx, 0)

    rT0 = (rT00, rT01)
    rT1 = (rT10, rT11)
    sw0 = (sw00, sw01)
    sw1 = (sw10, sw11)

    _prep_idx(idxall, idxg0, base, 0)
    pltpu.make_async_copy(table_hbm.at[idxg0], rows0, sg0).start()

    def pair(k, carry):
        ga = 2 * k

        _prep_idx(idxall, idxg1, base, ga + 1)
        pltpu.make_async_copy(table_hbm.at[idxg1], rows1, sg1).start()

        pltpu.make_async_copy(table_hbm.at[idxg0], rows0, sg0).wait()
        for j in range(G):
            @pl.when(k > 0)
            def _():
                _wait_writes(out_hbm, rT0[j], base, ga - 2, j, sw0[j])

            _transpose_unit(rows0, j, idxT, rT0[j])
            _start_writes(out_hbm, rT0[j], base, ga, j, sw0[j])

        @pl.when(k < GPW // 2 - 1)
        def _():
            _prep_idx(idxall, idxg0, base, ga + 2)
            pltpu.make_async_copy(table_hbm.at[idxg0], rows0, sg0).start()

        pltpu.make_async_copy(table_hbm.at[idxg1], rows1, sg1).wait()
        for j in range(G):
            @pl.when(k > 0)
            def _():
                _wait_writes(out_hbm, rT1[j], base, ga - 1, j, sw1[j])

            _transpose_unit(rows1, j, idxT, rT1[j])
            _start_writes(out_hbm, rT1[j], base, ga + 1, j, sw1[j])

        return carry

    lax.fori_loop(0, GPW // 2, pair, 0)

    last = GPW - 2
    for j in range(G):
        _wait_writes(out_hbm, rT0[j], base, last, j, sw0[j])
        _wait_writes(out_hbm, rT1[j], base, last + 1, j, sw1[j])


@jax.jit
def _lookup(xt, table):
    mesh = plsc.VectorSubcoreMesh(
        core_axis_name="c", subcore_axis_name="s", num_cores=NC, num_subcores=NS
    )
    return pl.kernel(
        _body,
        out_type=jax.ShapeDtypeStruct((F, 2, BT, 1024), jnp.float32),
        mesh=mesh,
        compiler_params=pltpu.CompilerParams(
            use_tc_tiling_on_sc=False, needs_layout_passes=False
        ),
        scratch_types=[
            pltpu.VMEM((UPW * 128,), jnp.int32),
            pltpu.VMEM((128 * LANES,), jnp.int32),
            pltpu.VMEM((G * 128,), jnp.int32),
            pltpu.VMEM((G * 128,), jnp.int32),
            pltpu.VMEM((G * 128, D), jnp.float32),
            pltpu.VMEM((G * 128, D), jnp.float32),
            pltpu.VMEM((D * 128,), jnp.float32),
            pltpu.VMEM((D * 128,), jnp.float32),
            pltpu.VMEM((D * 128,), jnp.float32),
            pltpu.VMEM((D * 128,), jnp.float32),
            pltpu.SemaphoreType.DMA,
            pltpu.SemaphoreType.DMA,
            pltpu.SemaphoreType.DMA,
            pltpu.SemaphoreType.DMA,
            pltpu.SemaphoreType.DMA,
            pltpu.SemaphoreType.DMA,
        ],
    )(xt, table)


def kernel(x, table):
    out5 = _lookup(x.T.reshape(F * B), table).reshape(F, 2, BT, 8, 128)
    return out5.transpose(2, 4, 0, 1, 3).reshape(B, F, D)
